# baseline (device time: 49338 ns/iter reference)
import os

import jax
import jax.numpy as jnp
from jax import lax
from jax.experimental import pallas as pl
from jax.experimental.pallas import tpu as pltpu

N_DEV = 16
_KVAR = os.environ.get("KVAR", "full")
_KN = int(os.environ.get("KN", "15"))
_KSZ = int(os.environ.get("KSZ", "1"))


def _gelu(y):
    c = 0.7978845608028654
    return 0.5 * y * (1.0 + jnp.tanh(c * (y + 0.044715 * y * y * y)))


def kernel(x, w_mat):
    m_per, k = x.shape
    _, n = w_mat.shape
    blk_n = n // N_DEV
    m_out = m_per * N_DEV

    comm = _KVAR in ("full", "comm")
    compute = _KVAR not in ("comm", "flows")

    def body(x_ref, w_ref, out_ref, yb_ref, send_sems, recv_sems, in_sems):
        my = lax.axis_index("i")

        if _KVAR == "nbr":
            barrier_sem = pltpu.get_barrier_semaphore()
            for nbr_d in (1, N_DEV - 1):
                peer = (my + nbr_d) % N_DEV
                pl.semaphore_signal(
                    barrier_sem, inc=1,
                    device_id=(peer,), device_id_type=pl.DeviceIdType.MESH,
                )
            pl.semaphore_wait(barrier_sem, 2)
            return

        if _KVAR == "flows":
            barrier_sem = pltpu.get_barrier_semaphore()
            for d in range(1, N_DEV):
                peer = (my + d) % N_DEV
                pl.semaphore_signal(
                    barrier_sem, inc=1,
                    device_id=(peer,), device_id_type=pl.DeviceIdType.MESH,
                )
            pl.semaphore_wait(barrier_sem, N_DEV - 1)
            flows = []
            for d in range(1, _KN + 1):
                dst = (my + d) % N_DEV
                rdma = pltpu.make_async_remote_copy(
                    src_ref=yb_ref.at[pl.ds(0, _KSZ)],
                    dst_ref=yb_ref.at[pl.ds(d * _KSZ, _KSZ)],
                    send_sem=send_sems.at[d],
                    recv_sem=recv_sems.at[d],
                    device_id=(dst,),
                    device_id_type=pl.DeviceIdType.MESH,
                )
                rdma.start()
                flows.append(rdma)
            for rdma in flows:
                rdma.wait()
            return

        if comm:
            barrier_sem = pltpu.get_barrier_semaphore()
            for nbr_d in (1, N_DEV - 1):
                peer = (my + nbr_d) % N_DEV
                pl.semaphore_signal(
                    barrier_sem, inc=1,
                    device_id=(peer,), device_id_type=pl.DeviceIdType.MESH,
                )
            for e in range(1, N_DEV):
                peer = (my + e) % N_DEV
                pl.semaphore_signal(
                    in_sems.at[N_DEV - e], inc=1,
                    device_id=(peer,), device_id_type=pl.DeviceIdType.MESH,
                )

        if compute:
            for d in range(1, N_DEV):
                dst = (my + d) % N_DEV
                yb_ref[d, :, :] = _gelu(
                    jnp.dot(x_ref[:, :], w_ref[:, pl.ds(dst * blk_n, blk_n)],
                            preferred_element_type=jnp.float32)
                )

        sends = []
        if comm:
            for d in range(1, N_DEV):
                dst = (my + d) % N_DEV
                pl.semaphore_wait(in_sems.at[d], 1)
                rdma = pltpu.make_async_remote_copy(
                    src_ref=yb_ref.at[d],
                    dst_ref=out_ref.at[pl.ds(my * m_per, m_per), :],
                    send_sem=send_sems.at[d],
                    recv_sem=recv_sems.at[d],
                    device_id=(dst,),
                    device_id_type=pl.DeviceIdType.MESH,
                )
                rdma.start()
                sends.append(rdma)

        if compute:
            out_ref[pl.ds(my * m_per, m_per), :] = _gelu(
                jnp.dot(x_ref[:, :], w_ref[:, pl.ds(my * blk_n, blk_n)],
                        preferred_element_type=jnp.float32)
            )

        if comm:
            for d in range(1, N_DEV):
                src = (my + N_DEV - d) % N_DEV
                recv = pltpu.make_async_remote_copy(
                    src_ref=yb_ref.at[d],
                    dst_ref=out_ref.at[pl.ds(src * m_per, m_per), :],
                    send_sem=send_sems.at[d],
                    recv_sem=recv_sems.at[d],
                    device_id=(my,),
                    device_id_type=pl.DeviceIdType.MESH,
                )
                recv.wait_recv()

            for rdma in sends:
                rdma.wait_send()

            pl.semaphore_wait(barrier_sem, 2)

    return pl.pallas_call(
        body,
        out_shape=jax.ShapeDtypeStruct((m_out, blk_n), jnp.float32),
        in_specs=[
            pl.BlockSpec(memory_space=pltpu.VMEM),
            pl.BlockSpec(memory_space=pltpu.VMEM),
        ],
        out_specs=pl.BlockSpec(memory_space=pltpu.VMEM),
        scratch_shapes=[
            pltpu.VMEM((N_DEV, m_per, blk_n), jnp.float32),
            pltpu.SemaphoreType.DMA((N_DEV,)),
            pltpu.SemaphoreType.DMA((N_DEV,)),
            pltpu.SemaphoreType.REGULAR((N_DEV,)),
        ],
        compiler_params=pltpu.CompilerParams(
            collective_id=0 if (comm or _KVAR in ("flows", "nbr")) else None,
            vmem_limit_bytes=100 * 1024 * 1024,
        ),
    )(x, w_mat)


# device time: 44453 ns/iter; 1.1099x vs baseline; 1.1099x over previous
import os

import jax
import jax.numpy as jnp
from jax import lax
from jax.experimental import pallas as pl
from jax.experimental.pallas import tpu as pltpu

N_DEV = 16
_KVAR = os.environ.get("KVAR", "full")
_KN = int(os.environ.get("KN", "15"))
_KSZ = int(os.environ.get("KSZ", "1"))


def _gelu(y):
    c = 0.7978845608028654
    return 0.5 * y * (1.0 + jnp.tanh(c * (y + 0.044715 * y * y * y)))


def kernel(x, w_mat):
    m_per, k = x.shape
    _, n = w_mat.shape
    blk_n = n // N_DEV
    m_out = m_per * N_DEV

    comm = _KVAR in ("full", "comm")
    compute = _KVAR not in ("comm", "flows")

    def body(x_ref, w_ref, out_ref, yb_ref, xb_ref, wb_ref,
             send_sems, recv_sems, in_sems):
        my = lax.axis_index("i")

        if _KVAR == "nbr":
            barrier_sem = pltpu.get_barrier_semaphore()
            for nbr_d in (1, N_DEV - 1):
                peer = (my + nbr_d) % N_DEV
                pl.semaphore_signal(
                    barrier_sem, inc=1,
                    device_id=(peer,), device_id_type=pl.DeviceIdType.MESH,
                )
            pl.semaphore_wait(barrier_sem, 2)
            return

        if _KVAR == "flows":
            barrier_sem = pltpu.get_barrier_semaphore()
            for d in range(1, N_DEV):
                peer = (my + d) % N_DEV
                pl.semaphore_signal(
                    barrier_sem, inc=1,
                    device_id=(peer,), device_id_type=pl.DeviceIdType.MESH,
                )
            pl.semaphore_wait(barrier_sem, N_DEV - 1)
            flows = []
            for d in range(1, _KN + 1):
                dst = (my + d) % N_DEV
                rdma = pltpu.make_async_remote_copy(
                    src_ref=yb_ref.at[pl.ds(0, _KSZ)],
                    dst_ref=yb_ref.at[pl.ds(d * _KSZ, _KSZ)],
                    send_sem=send_sems.at[d],
                    recv_sem=recv_sems.at[d],
                    device_id=(dst,),
                    device_id_type=pl.DeviceIdType.MESH,
                )
                rdma.start()
                flows.append(rdma)
            for rdma in flows:
                rdma.wait()
            return

        if comm:
            barrier_sem = pltpu.get_barrier_semaphore()
            for nbr_d in (1, N_DEV - 1):
                peer = (my + nbr_d) % N_DEV
                pl.semaphore_signal(
                    barrier_sem, inc=1,
                    device_id=(peer,), device_id_type=pl.DeviceIdType.MESH,
                )
            for e in range(1, N_DEV):
                peer = (my + e) % N_DEV
                pl.semaphore_signal(
                    in_sems.at[N_DEV - e], inc=1,
                    device_id=(peer,), device_id_type=pl.DeviceIdType.MESH,
                )

        if compute:
            xb_ref[:, :] = x_ref[:, :].astype(jnp.bfloat16)
            wb_ref[:, :] = w_ref[:, :].astype(jnp.bfloat16)

        sends = []
        for d in range(1, N_DEV):
            dst = (my + d) % N_DEV
            if compute:
                yb_ref[d, :, :] = _gelu(
                    jnp.dot(xb_ref[:, :], wb_ref[:, pl.ds(dst * blk_n, blk_n)],
                            preferred_element_type=jnp.float32)
                )
            if comm:
                pl.semaphore_wait(in_sems.at[d], 1)
                rdma = pltpu.make_async_remote_copy(
                    src_ref=yb_ref.at[d],
                    dst_ref=out_ref.at[pl.ds(my * m_per, m_per), :],
                    send_sem=send_sems.at[d],
                    recv_sem=recv_sems.at[d],
                    device_id=(dst,),
                    device_id_type=pl.DeviceIdType.MESH,
                )
                rdma.start()
                sends.append(rdma)

        if compute:
            out_ref[pl.ds(my * m_per, m_per), :] = _gelu(
                jnp.dot(xb_ref[:, :], wb_ref[:, pl.ds(my * blk_n, blk_n)],
                        preferred_element_type=jnp.float32)
            )

        if comm:
            for d in range(1, N_DEV):
                src = (my + N_DEV - d) % N_DEV
                recv = pltpu.make_async_remote_copy(
                    src_ref=yb_ref.at[d],
                    dst_ref=out_ref.at[pl.ds(src * m_per, m_per), :],
                    send_sem=send_sems.at[d],
                    recv_sem=recv_sems.at[d],
                    device_id=(my,),
                    device_id_type=pl.DeviceIdType.MESH,
                )
                recv.wait_recv()

            for rdma in sends:
                rdma.wait_send()

            pl.semaphore_wait(barrier_sem, 2)

    return pl.pallas_call(
        body,
        out_shape=jax.ShapeDtypeStruct((m_out, blk_n), jnp.float32),
        in_specs=[
            pl.BlockSpec(memory_space=pltpu.VMEM),
            pl.BlockSpec(memory_space=pltpu.VMEM),
        ],
        out_specs=pl.BlockSpec(memory_space=pltpu.VMEM),
        scratch_shapes=[
            pltpu.VMEM((N_DEV, m_per, blk_n), jnp.float32),
            pltpu.VMEM((m_per, k), jnp.bfloat16),
            pltpu.VMEM((k, n), jnp.bfloat16),
            pltpu.SemaphoreType.DMA((N_DEV,)),
            pltpu.SemaphoreType.DMA((N_DEV,)),
            pltpu.SemaphoreType.REGULAR((N_DEV,)),
        ],
        compiler_params=pltpu.CompilerParams(
            collective_id=0 if (comm or _KVAR in ("flows", "nbr")) else None,
            vmem_limit_bytes=100 * 1024 * 1024,
        ),
    )(x, w_mat)


# device time: 44126 ns/iter; 1.1181x vs baseline; 1.0074x over previous
import os

import jax
import jax.numpy as jnp
from jax import lax
from jax.experimental import pallas as pl
from jax.experimental.pallas import tpu as pltpu

N_DEV = 16
_KVAR = os.environ.get("KVAR", "full")
_KN = int(os.environ.get("KN", "15"))
_KSZ = int(os.environ.get("KSZ", "1"))


def _gelu(y):
    c = 0.7978845608028654
    return 0.5 * y * (1.0 + jnp.tanh(c * (y + 0.044715 * y * y * y)))


def kernel(x, w_mat):
    m_per, k = x.shape
    _, n = w_mat.shape
    blk_n = n // N_DEV
    m_out = m_per * N_DEV

    comm = _KVAR in ("full", "comm")
    compute = _KVAR not in ("comm", "flows")

    def body(x_ref, w_ref, out_ref, y_ref, send_sems, recv_sems, in_sems):
        my = lax.axis_index("i")

        if _KVAR == "nbr":
            barrier_sem = pltpu.get_barrier_semaphore()
            for nbr_d in (1, N_DEV - 1):
                peer = (my + nbr_d) % N_DEV
                pl.semaphore_signal(
                    barrier_sem, inc=1,
                    device_id=(peer,), device_id_type=pl.DeviceIdType.MESH,
                )
            pl.semaphore_wait(barrier_sem, 2)
            return

        if _KVAR == "flows":
            barrier_sem = pltpu.get_barrier_semaphore()
            for d in range(1, N_DEV):
                peer = (my + d) % N_DEV
                pl.semaphore_signal(
                    barrier_sem, inc=1,
                    device_id=(peer,), device_id_type=pl.DeviceIdType.MESH,
                )
            pl.semaphore_wait(barrier_sem, N_DEV - 1)
            flows = []
            for d in range(1, _KN + 1):
                dst = (my + d) % N_DEV
                rdma = pltpu.make_async_remote_copy(
                    src_ref=y_ref.at[:, pl.ds(0, _KSZ * blk_n)],
                    dst_ref=y_ref.at[:, pl.ds((d % (N_DEV // _KSZ)) * _KSZ * blk_n, _KSZ * blk_n)],
                    send_sem=send_sems.at[d],
                    recv_sem=recv_sems.at[d],
                    device_id=(dst,),
                    device_id_type=pl.DeviceIdType.MESH,
                )
                rdma.start()
                flows.append(rdma)
            for rdma in flows:
                rdma.wait()
            return

        if comm:
            barrier_sem = pltpu.get_barrier_semaphore()
            for nbr_d in (1, N_DEV - 1):
                peer = (my + nbr_d) % N_DEV
                pl.semaphore_signal(
                    barrier_sem, inc=1,
                    device_id=(peer,), device_id_type=pl.DeviceIdType.MESH,
                )
            for e in range(1, N_DEV):
                peer = (my + e) % N_DEV
                pl.semaphore_signal(
                    in_sems.at[N_DEV - e], inc=1,
                    device_id=(peer,), device_id_type=pl.DeviceIdType.MESH,
                )

        if compute:
            y_ref[:, :] = _gelu(
                jnp.dot(x_ref[:, :], w_ref[:, :],
                        preferred_element_type=jnp.float32)
            )

        sends = []
        if comm:
            for d in range(1, N_DEV):
                dst = (my + d) % N_DEV
                pl.semaphore_wait(in_sems.at[d], 1)
                rdma = pltpu.make_async_remote_copy(
                    src_ref=y_ref.at[:, pl.ds(dst * blk_n, blk_n)],
                    dst_ref=out_ref.at[pl.ds(my * m_per, m_per), :],
                    send_sem=send_sems.at[d],
                    recv_sem=recv_sems.at[d],
                    device_id=(dst,),
                    device_id_type=pl.DeviceIdType.MESH,
                )
                rdma.start()
                sends.append(rdma)

        if compute:
            out_ref[pl.ds(my * m_per, m_per), :] = y_ref[:, pl.ds(my * blk_n, blk_n)]

        if comm:
            for d in range(1, N_DEV):
                src = (my + N_DEV - d) % N_DEV
                recv = pltpu.make_async_remote_copy(
                    src_ref=y_ref.at[:, pl.ds(0, blk_n)],
                    dst_ref=out_ref.at[pl.ds(src * m_per, m_per), :],
                    send_sem=send_sems.at[d],
                    recv_sem=recv_sems.at[d],
                    device_id=(my,),
                    device_id_type=pl.DeviceIdType.MESH,
                )
                recv.wait_recv()

            for rdma in sends:
                rdma.wait_send()

            pl.semaphore_wait(barrier_sem, 2)

    return pl.pallas_call(
        body,
        out_shape=jax.ShapeDtypeStruct((m_out, blk_n), jnp.float32),
        in_specs=[
            pl.BlockSpec(memory_space=pltpu.VMEM),
            pl.BlockSpec(memory_space=pltpu.VMEM),
        ],
        out_specs=pl.BlockSpec(memory_space=pltpu.VMEM),
        scratch_shapes=[
            pltpu.VMEM((m_per, n), jnp.float32),
            pltpu.SemaphoreType.DMA((N_DEV,)),
            pltpu.SemaphoreType.DMA((N_DEV,)),
            pltpu.SemaphoreType.REGULAR((N_DEV,)),
        ],
        compiler_params=pltpu.CompilerParams(
            collective_id=0 if (comm or _KVAR in ("flows", "nbr")) else None,
            vmem_limit_bytes=100 * 1024 * 1024,
        ),
    )(x, w_mat)
